# baseline (speedup 1.0000x reference)
import jax
import jax.numpy as jnp
from jax.experimental import pallas as pl
from jax.experimental.pallas import tpu as pltpu

_SIZE = 2048
_BLOCK_ROWS = 512


def _body(sel_ref, rm_ref, data_ref, out_ref):
    mask = rm_ref[0].astype(jnp.float32)
    out_ref[...] = data_ref[...] * mask


def kernel(data, region_map, selected_param):
    n_rows = data.shape[0]
    sel = jnp.asarray(selected_param, jnp.int32).reshape(1)
    rm3 = region_map.reshape(region_map.shape[0], 1, _SIZE)
    grid = (n_rows // _BLOCK_ROWS,)
    return pl.pallas_call(
        _body,
        grid_spec=pltpu.PrefetchScalarGridSpec(
            num_scalar_prefetch=1,
            grid=grid,
            in_specs=[
                pl.BlockSpec((1, 1, _SIZE), lambda i, sel: (sel[0], 0, 0)),
                pl.BlockSpec((_BLOCK_ROWS, _SIZE), lambda i, sel: (i, 0)),
            ],
            out_specs=pl.BlockSpec((_BLOCK_ROWS, _SIZE), lambda i, sel: (i, 0)),
        ),
        out_shape=jax.ShapeDtypeStruct((n_rows, _SIZE), jnp.float32),
        compiler_params=pltpu.CompilerParams(
            dimension_semantics=("arbitrary",),
        ),
    )(sel, rm3, data)


# speedup vs baseline: 1.0338x; 1.0338x over previous
import jax
import jax.numpy as jnp
from jax.experimental import pallas as pl
from jax.experimental.pallas import tpu as pltpu

_SIZE = 2048
_BLOCK_ROWS = 1024


def _body(sel_ref, rm_ref, data_ref, out_ref):
    mask = rm_ref[0].astype(jnp.float32)
    out_ref[...] = data_ref[...] * mask


def kernel(data, region_map, selected_param):
    n_rows = data.shape[0]
    sel = jnp.asarray(selected_param, jnp.int32).reshape(1)
    rm3 = region_map.reshape(region_map.shape[0], 1, _SIZE)
    grid = (n_rows // _BLOCK_ROWS,)
    return pl.pallas_call(
        _body,
        grid_spec=pltpu.PrefetchScalarGridSpec(
            num_scalar_prefetch=1,
            grid=grid,
            in_specs=[
                pl.BlockSpec((1, 1, _SIZE), lambda i, sel: (sel[0], 0, 0)),
                pl.BlockSpec((_BLOCK_ROWS, _SIZE), lambda i, sel: (i, 0)),
            ],
            out_specs=pl.BlockSpec((_BLOCK_ROWS, _SIZE), lambda i, sel: (i, 0)),
        ),
        out_shape=jax.ShapeDtypeStruct((n_rows, _SIZE), jnp.float32),
        compiler_params=pltpu.CompilerParams(
            dimension_semantics=("arbitrary",),
        ),
    )(sel, rm3, data)
